# baseline (speedup 1.0000x reference)
import functools

import jax
import jax.numpy as jnp
from jax import lax
from jax.experimental import pallas as pl
from jax.experimental.pallas import tpu as pltpu
from jax.experimental.pallas import tpu_sc as plsc

N_NODES = 10000
N_EDGES = 320000
D_FEAT = 128

NC = 2
NS = 16
NW = NC * NS
L = 16

CH = 128
N_CHUNK = 80
E_PER_W = N_CHUNK * CH
E_PAD = NW * E_PER_W
N_PAD = N_NODES + 240
ROWS_PER_TILE = N_PAD // NS

A_CH = 48
B_CH = 2 * N_CHUNK - A_CH
MAXC = max(A_CH, B_CH)
TOT_CH = NW * N_CHUNK
TOT_CH_PAD = TOT_CH + MAXC

_mesh = plsc.VectorSubcoreMesh(core_axis_name="c", subcore_axis_name="s",
                               num_cores=NC, num_subcores=NS)


@functools.partial(
    pl.kernel,
    out_type=jax.ShapeDtypeStruct((NC, N_PAD, D_FEAT), jnp.float32),
    mesh=_mesh,
    scratch_types=[
        pltpu.VMEM_SHARED((N_PAD, D_FEAT), jnp.float32),
        pltpu.VMEM((N_CHUNK, CH), jnp.int32),
        pltpu.VMEM((CH, D_FEAT), jnp.float32),
        pltpu.VMEM((L, D_FEAT), jnp.float32),
    ],
)
def _deg_kernel(dst_hbm, deg_out, hist, dst_v, ones_v, zbuf):
    c = lax.axis_index("c")
    s = lax.axis_index("s")
    w = c * NS + s

    pltpu.sync_copy(dst_hbm.at[pl.ds(w * N_CHUNK, N_CHUNK)], dst_v)

    ones = jnp.ones((L,), jnp.float32)
    zeros = jnp.zeros((L,), jnp.float32)

    @pl.loop(0, CH)
    def _(i):
        for j in range(D_FEAT // L):
            ones_v[i, pl.ds(j * L, L)] = ones

    for i in range(L):
        for j in range(D_FEAT // L):
            zbuf[i, pl.ds(j * L, L)] = zeros

    row0 = s * ROWS_PER_TILE

    @pl.loop(0, ROWS_PER_TILE // L)
    def _(r):
        pltpu.sync_copy(zbuf, hist.at[pl.ds(row0 + r * L, L)])
    plsc.subcore_barrier()

    @pl.loop(0, N_CHUNK)
    def _(j):
        pltpu.sync_copy(ones_v, hist.at[dst_v.at[j]], add=True)

    plsc.subcore_barrier()
    pltpu.sync_copy(hist.at[pl.ds(row0, ROWS_PER_TILE)],
                    deg_out.at[c, pl.ds(row0, ROWS_PER_TILE)])


_TC_BLK = N_NODES // 10


def _dis_block(dp_ref):
    deg = dp_ref[0, :, 0] + dp_ref[1, :, 0]
    return jnp.where(deg > 0.0, lax.rsqrt(deg), 0.0)


def _norm_body(dp_ref, x_ref, xp_ref):
    dis = _dis_block(dp_ref)
    xp_ref[...] = x_ref[...] * dis[:, None]


def _norm_scale(deg_part, x):
    return pl.pallas_call(
        _norm_body,
        grid=(10,),
        in_specs=[
            pl.BlockSpec((NC, _TC_BLK, D_FEAT), lambda i: (0, i, 0)),
            pl.BlockSpec((_TC_BLK, D_FEAT), lambda i: (i, 0)),
        ],
        out_specs=pl.BlockSpec((_TC_BLK, D_FEAT), lambda i: (i, 0)),
        out_shape=jax.ShapeDtypeStruct((N_NODES, D_FEAT), jnp.float32),
    )(deg_part, x)


@functools.partial(
    pl.kernel,
    out_type=jax.ShapeDtypeStruct((NC, N_PAD, D_FEAT), jnp.float32),
    mesh=_mesh,
    scratch_types=[
        pltpu.VMEM_SHARED((N_PAD, D_FEAT), jnp.float32),
        pltpu.VMEM((MAXC, CH), jnp.int32),
        pltpu.VMEM((2, CH), jnp.int32),
        [pltpu.VMEM((CH, D_FEAT), jnp.float32) for _ in range(2)],
        [pltpu.SemaphoreType.DMA for _ in range(2)],
        [pltpu.SemaphoreType.DMA for _ in range(2)],
        [pltpu.SemaphoreType.DMA for _ in range(2)],
    ],
)
def _scatter_kernel(xp_hbm, src_hbm, dst_hbm, out_hbm,
                    acc, src_v, dring, rows, gsem, ssem, dsem):
    c = lax.axis_index("c")
    s = lax.axis_index("s")
    nch = jnp.where(c == 0, A_CH, B_CH)
    base = jnp.where(c == 0, s * A_CH, NS * A_CH + s * B_CH)

    pltpu.sync_copy(src_hbm.at[pl.ds(base, MAXC)], src_v)

    zeros = jnp.zeros((L,), jnp.float32)
    for i in range(L):
        for j in range(D_FEAT // L):
            rows[0][i, pl.ds(j * L, L)] = zeros
    row0 = s * ROWS_PER_TILE

    @pl.loop(0, ROWS_PER_TILE // L)
    def _(r):
        pltpu.sync_copy(rows[0].at[pl.ds(0, L)],
                        acc.at[pl.ds(row0 + r * L, L)])
    plsc.subcore_barrier()

    for b in range(2):
        pltpu.async_copy(dst_hbm.at[base + b], dring.at[b], dsem[b])
        pltpu.async_copy(xp_hbm.at[src_v.at[b]], rows[b], gsem[b])

    def step(j, b, first):
        o = 1 - b

        def retire():
            pltpu.make_async_copy(rows[o], acc.at[dring.at[o]],
                                  ssem[o]).wait()

            @pl.when(j + 1 < nch)
            def _():
                pltpu.async_copy(dst_hbm.at[base + j + 1], dring.at[o],
                                 dsem[o])
                pltpu.async_copy(xp_hbm.at[src_v.at[j + 1]], rows[o], gsem[o])

        if first:
            pl.when(j >= 1)(retire)
        else:
            retire()

        pltpu.make_async_copy(dst_hbm.at[base + j], dring.at[b],
                              dsem[b]).wait()
        pltpu.make_async_copy(xp_hbm.at[src_v.at[j]], rows[b], gsem[b]).wait()
        pltpu.async_copy(rows[b], acc.at[dring.at[b]], ssem[b], add=True)

    @pl.loop(0, nch // 2)
    def _(g):
        j0 = g * 2
        step(j0, 0, True)
        step(j0 + 1, 1, False)

    pltpu.make_async_copy(rows[1], acc.at[dring.at[1]], ssem[1]).wait()

    plsc.subcore_barrier()
    pltpu.sync_copy(acc.at[pl.ds(row0, ROWS_PER_TILE)],
                    out_hbm.at[c, pl.ds(row0, ROWS_PER_TILE)])


def _combine_body(part_ref, dp_ref, out_ref):
    dis = _dis_block(dp_ref)
    out_ref[...] = (part_ref[0] + part_ref[1]) * dis[:, None]


def _combine(part, deg_part):
    return pl.pallas_call(
        _combine_body,
        grid=(10,),
        in_specs=[
            pl.BlockSpec((NC, _TC_BLK, D_FEAT), lambda i: (0, i, 0)),
            pl.BlockSpec((NC, _TC_BLK, D_FEAT), lambda i: (0, i, 0)),
        ],
        out_specs=pl.BlockSpec((_TC_BLK, D_FEAT), lambda i: (i, 0)),
        out_shape=jax.ShapeDtypeStruct((N_NODES, D_FEAT), jnp.float32),
    )(part, deg_part)


def kernel(taste_x, taste_edge_index):
    ei = taste_edge_index.astype(jnp.int32)
    pad = TOT_CH_PAD * CH - N_EDGES
    src = jnp.concatenate([ei[0], jnp.zeros((pad,), jnp.int32)])
    dst = jnp.concatenate([ei[1], jnp.full((pad,), N_NODES, jnp.int32)])
    src = src.reshape(TOT_CH_PAD, CH)
    dst = dst.reshape(TOT_CH_PAD, CH)
    deg_part = _deg_kernel(dst)
    xp = _norm_scale(deg_part, taste_x)
    part = _scatter_kernel(xp, src, dst)
    return _combine(part, deg_part)


# speedup vs baseline: 1.0572x; 1.0572x over previous
import functools

import jax
import jax.numpy as jnp
from jax import lax
from jax.experimental import pallas as pl
from jax.experimental.pallas import tpu as pltpu
from jax.experimental.pallas import tpu_sc as plsc

N_NODES = 10000
N_EDGES = 320000
D_FEAT = 128

NC = 2
NS = 16
NW = NC * NS
L = 16

CH = 128
N_CHUNK = 80
E_PER_W = N_CHUNK * CH
E_PAD = NW * E_PER_W
N_PAD = N_NODES + 240
ROWS_PER_TILE = N_PAD // NS

A_CH = 112
B_CH = 2 * N_CHUNK - A_CH
MAXC = max(A_CH, B_CH)
TOT_CH = NW * N_CHUNK
TOT_CH_PAD = TOT_CH + MAXC

_mesh = plsc.VectorSubcoreMesh(core_axis_name="c", subcore_axis_name="s",
                               num_cores=NC, num_subcores=NS)


@functools.partial(
    pl.kernel,
    out_type=jax.ShapeDtypeStruct((NC, N_PAD, D_FEAT), jnp.float32),
    mesh=_mesh,
    scratch_types=[
        pltpu.VMEM_SHARED((N_PAD, D_FEAT), jnp.float32),
        pltpu.VMEM((N_CHUNK, CH), jnp.int32),
        pltpu.VMEM((CH, D_FEAT), jnp.float32),
        pltpu.VMEM((L, D_FEAT), jnp.float32),
    ],
)
def _deg_kernel(dst_hbm, deg_out, hist, dst_v, ones_v, zbuf):
    c = lax.axis_index("c")
    s = lax.axis_index("s")
    w = c * NS + s

    pltpu.sync_copy(dst_hbm.at[pl.ds(w * N_CHUNK, N_CHUNK)], dst_v)

    ones = jnp.ones((L,), jnp.float32)
    zeros = jnp.zeros((L,), jnp.float32)

    @pl.loop(0, CH)
    def _(i):
        for j in range(D_FEAT // L):
            ones_v[i, pl.ds(j * L, L)] = ones

    for i in range(L):
        for j in range(D_FEAT // L):
            zbuf[i, pl.ds(j * L, L)] = zeros

    row0 = s * ROWS_PER_TILE

    @pl.loop(0, ROWS_PER_TILE // L)
    def _(r):
        pltpu.sync_copy(zbuf, hist.at[pl.ds(row0 + r * L, L)])
    plsc.subcore_barrier()

    @pl.loop(0, N_CHUNK)
    def _(j):
        pltpu.sync_copy(ones_v, hist.at[dst_v.at[j]], add=True)

    plsc.subcore_barrier()
    pltpu.sync_copy(hist.at[pl.ds(row0, ROWS_PER_TILE)],
                    deg_out.at[c, pl.ds(row0, ROWS_PER_TILE)])


_TC_BLK = N_NODES // 10


def _dis_block(dp_ref):
    deg = dp_ref[0, :, 0] + dp_ref[1, :, 0]
    return jnp.where(deg > 0.0, lax.rsqrt(deg), 0.0)


def _norm_body(dp_ref, x_ref, xp_ref):
    dis = _dis_block(dp_ref)
    xp_ref[...] = x_ref[...] * dis[:, None]


def _norm_scale(deg_part, x):
    return pl.pallas_call(
        _norm_body,
        grid=(10,),
        in_specs=[
            pl.BlockSpec((NC, _TC_BLK, D_FEAT), lambda i: (0, i, 0)),
            pl.BlockSpec((_TC_BLK, D_FEAT), lambda i: (i, 0)),
        ],
        out_specs=pl.BlockSpec((_TC_BLK, D_FEAT), lambda i: (i, 0)),
        out_shape=jax.ShapeDtypeStruct((N_NODES, D_FEAT), jnp.float32),
    )(deg_part, x)


@functools.partial(
    pl.kernel,
    out_type=jax.ShapeDtypeStruct((NC, N_PAD, D_FEAT), jnp.float32),
    mesh=_mesh,
    scratch_types=[
        pltpu.VMEM_SHARED((N_PAD, D_FEAT), jnp.float32),
        pltpu.VMEM((MAXC, CH), jnp.int32),
        pltpu.VMEM((2, CH), jnp.int32),
        [pltpu.VMEM((CH, D_FEAT), jnp.float32) for _ in range(2)],
        [pltpu.SemaphoreType.DMA for _ in range(2)],
        [pltpu.SemaphoreType.DMA for _ in range(2)],
        [pltpu.SemaphoreType.DMA for _ in range(2)],
    ],
)
def _scatter_kernel(xp_hbm, src_hbm, dst_hbm, out_hbm,
                    acc, src_v, dring, rows, gsem, ssem, dsem):
    c = lax.axis_index("c")
    s = lax.axis_index("s")
    nch = jnp.where(c == 0, A_CH, B_CH)
    base = jnp.where(c == 0, s * A_CH, NS * A_CH + s * B_CH)

    pltpu.sync_copy(src_hbm.at[pl.ds(base, MAXC)], src_v)

    zeros = jnp.zeros((L,), jnp.float32)
    for i in range(L):
        for j in range(D_FEAT // L):
            rows[0][i, pl.ds(j * L, L)] = zeros
    row0 = s * ROWS_PER_TILE

    @pl.loop(0, ROWS_PER_TILE // L)
    def _(r):
        pltpu.sync_copy(rows[0].at[pl.ds(0, L)],
                        acc.at[pl.ds(row0 + r * L, L)])
    plsc.subcore_barrier()

    for b in range(2):
        pltpu.async_copy(dst_hbm.at[base + b], dring.at[b], dsem[b])
        pltpu.async_copy(xp_hbm.at[src_v.at[b]], rows[b], gsem[b])

    def step(j, b, first):
        o = 1 - b

        def retire():
            pltpu.make_async_copy(rows[o], acc.at[dring.at[o]],
                                  ssem[o]).wait()

            @pl.when(j + 1 < nch)
            def _():
                pltpu.async_copy(dst_hbm.at[base + j + 1], dring.at[o],
                                 dsem[o])
                pltpu.async_copy(xp_hbm.at[src_v.at[j + 1]], rows[o], gsem[o])

        if first:
            pl.when(j >= 1)(retire)
        else:
            retire()

        pltpu.make_async_copy(dst_hbm.at[base + j], dring.at[b],
                              dsem[b]).wait()
        pltpu.make_async_copy(xp_hbm.at[src_v.at[j]], rows[b], gsem[b]).wait()
        pltpu.async_copy(rows[b], acc.at[dring.at[b]], ssem[b], add=True)

    @pl.loop(0, nch // 2)
    def _(g):
        j0 = g * 2
        step(j0, 0, True)
        step(j0 + 1, 1, False)

    pltpu.make_async_copy(rows[1], acc.at[dring.at[1]], ssem[1]).wait()

    plsc.subcore_barrier()
    pltpu.sync_copy(acc.at[pl.ds(row0, ROWS_PER_TILE)],
                    out_hbm.at[c, pl.ds(row0, ROWS_PER_TILE)])


def _combine_body(part_ref, dp_ref, out_ref):
    dis = _dis_block(dp_ref)
    out_ref[...] = (part_ref[0] + part_ref[1]) * dis[:, None]


def _combine(part, deg_part):
    return pl.pallas_call(
        _combine_body,
        grid=(10,),
        in_specs=[
            pl.BlockSpec((NC, _TC_BLK, D_FEAT), lambda i: (0, i, 0)),
            pl.BlockSpec((NC, _TC_BLK, D_FEAT), lambda i: (0, i, 0)),
        ],
        out_specs=pl.BlockSpec((_TC_BLK, D_FEAT), lambda i: (i, 0)),
        out_shape=jax.ShapeDtypeStruct((N_NODES, D_FEAT), jnp.float32),
    )(part, deg_part)


def kernel(taste_x, taste_edge_index):
    ei = taste_edge_index.astype(jnp.int32)
    pad = TOT_CH_PAD * CH - N_EDGES
    src = jnp.concatenate([ei[0], jnp.zeros((pad,), jnp.int32)])
    dst = jnp.concatenate([ei[1], jnp.full((pad,), N_NODES, jnp.int32)])
    src = src.reshape(TOT_CH_PAD, CH)
    dst = dst.reshape(TOT_CH_PAD, CH)
    deg_part = _deg_kernel(dst)
    xp = _norm_scale(deg_part, taste_x)
    part = _scatter_kernel(xp, src, dst)
    return _combine(part, deg_part)
